# Initial kernel scaffold; baseline (speedup 1.0000x reference)
import jax
import jax.numpy as jnp
from jax.experimental import pallas as pl


def kernel(x, transform_tnet3_conv1_w, transform_tnet3_conv1_scale, transform_tnet3_conv1_shift, transform_tnet3_conv2_w, transform_tnet3_conv2_scale, transform_tnet3_conv2_shift, transform_tnet3_conv3_w, transform_tnet3_conv3_scale, transform_tnet3_conv3_shift, transform_tnet3_fc1_w, transform_tnet3_fc1_scale, transform_tnet3_fc1_shift, transform_tnet3_fc2_w, transform_tnet3_fc2_scale, transform_tnet3_fc2_shift, transform_tnet3_fc3_w, transform_tnet3_fc3_scale, transform_tnet3_fc3_shift, transform_tnet64_conv1_w, transform_tnet64_conv1_scale, transform_tnet64_conv1_shift, transform_tnet64_conv2_w, transform_tnet64_conv2_scale, transform_tnet64_conv2_shift, transform_tnet64_conv3_w, transform_tnet64_conv3_scale, transform_tnet64_conv3_shift, transform_tnet64_fc1_w, transform_tnet64_fc1_scale, transform_tnet64_fc1_shift, transform_tnet64_fc2_w, transform_tnet64_fc2_scale, transform_tnet64_fc2_shift, transform_tnet64_fc3_w, transform_tnet64_fc3_scale, transform_tnet64_fc3_shift, transform_conv1_w, transform_conv1_scale, transform_conv1_shift, transform_conv2_w, transform_conv2_scale, transform_conv2_shift, transform_conv3_w, transform_conv3_scale, transform_conv3_shift, fc1_w, fc1_scale, fc1_shift, fc2_w, fc2_scale, fc2_shift, fc3_w, fc3_scale, fc3_shift):
    raise NotImplementedError("write your pallas kernel here")


# speedup vs baseline: 2.2419x; 2.2419x over previous
import functools

import jax
import jax.numpy as jnp
from jax import lax
from jax.experimental import pallas as pl
from jax.experimental.pallas import tpu as pltpu

_TN = 2048


def _bf16(w):
    return w.astype(jnp.bfloat16)


def _mlp_max_kernel(x_ref, *refs, per_batch, relu, n_layers):
    *prefs, o_ref, acc_ref = refs
    t = pl.program_id(1)

    @pl.when(t == 0)
    def _():
        acc_ref[...] = jnp.full(acc_ref.shape, -jnp.inf, jnp.float32)

    xc = x_ref[0].astype(jnp.bfloat16)
    h = None
    for i in range(n_layers):
        w_ref, s_ref, b_ref = prefs[3 * i], prefs[3 * i + 1], prefs[3 * i + 2]
        w = w_ref[0] if per_batch[i] else w_ref[...]
        if i == 0:
            h = lax.dot_general(xc, w, (((0,), (0,)), ((), ())),
                                preferred_element_type=jnp.float32)
        else:
            h = jnp.dot(h.astype(jnp.bfloat16), w,
                        preferred_element_type=jnp.float32)
        h = h * s_ref[...] + b_ref[...]
        if relu[i]:
            h = jnp.maximum(h, 0.0)

    part = jnp.max(h.reshape(-1, 8, h.shape[-1]), axis=0)
    acc_ref[...] = jnp.maximum(acc_ref[...], part)

    @pl.when(t == pl.num_programs(1) - 1)
    def _():
        o_ref[0] = jnp.max(acc_ref[...], axis=0, keepdims=True)


def _mlp_maxpool(x_bcn, layers, relus):
    B, _, N = x_bcn.shape
    tn = min(N, _TN)
    c_last = layers[-1][0].shape[-1]
    per_batch = tuple(w.ndim == 3 for (w, _, _) in layers)

    in_specs = [pl.BlockSpec((1, 3, tn), lambda b, t: (b, 0, t))]
    args = [x_bcn]
    for w, s, sh in layers:
        wb = _bf16(w)
        if wb.ndim == 3:
            in_specs.append(pl.BlockSpec((1,) + wb.shape[1:],
                                         lambda b, t: (b, 0, 0)))
        else:
            in_specs.append(pl.BlockSpec(wb.shape, lambda b, t: (0, 0)))
        in_specs.append(pl.BlockSpec(s.shape, lambda b, t: (0, 0)))
        in_specs.append(pl.BlockSpec(sh.shape, lambda b, t: (0, 0)))
        args += [wb, s, sh]

    out = pl.pallas_call(
        functools.partial(_mlp_max_kernel, per_batch=per_batch,
                          relu=tuple(relus), n_layers=len(layers)),
        out_shape=jax.ShapeDtypeStruct((B, 1, c_last), jnp.float32),
        grid=(B, N // tn),
        in_specs=in_specs,
        out_specs=pl.BlockSpec((1, 1, c_last), lambda b, t: (b, 0, 0)),
        scratch_shapes=[pltpu.VMEM((8, c_last), jnp.float32)],
        compiler_params=pltpu.CompilerParams(
            dimension_semantics=("parallel", "arbitrary")),
    )(*args)
    return out.reshape(B, c_last)


def _head_kernel(x_ref, w1, s1, b1, w2, s2, b2, w3, s3, b3, o_ref, *, logsm):
    h = jnp.dot(x_ref[...].astype(jnp.bfloat16), w1[...],
                preferred_element_type=jnp.float32)
    h = jnp.maximum(h * s1[...] + b1[...], 0.0)
    h = jnp.dot(h.astype(jnp.bfloat16), w2[...],
                preferred_element_type=jnp.float32)
    h = jnp.maximum(h * s2[...] + b2[...], 0.0)
    z = jnp.dot(h.astype(jnp.bfloat16), w3[...],
                preferred_element_type=jnp.float32)
    z = z * s3[...] + b3[...]
    if logsm:
        z = z - jnp.max(z, axis=-1, keepdims=True)
        z = z - jnp.log(jnp.sum(jnp.exp(z), axis=-1, keepdims=True))
    o_ref[...] = z


def _head(pooled, l1, l2, l3, *, logsm):
    B = pooled.shape[0]
    K = l3[0].shape[-1]
    full = lambda shape: pl.BlockSpec(shape, lambda: (0,) * len(shape))
    args, specs = [pooled], [full(pooled.shape)]
    for w, s, sh in (l1, l2, l3):
        wb = _bf16(w)
        args += [wb, s, sh]
        specs += [full(wb.shape), full(s.shape), full(sh.shape)]
    return pl.pallas_call(
        functools.partial(_head_kernel, logsm=logsm),
        out_shape=jax.ShapeDtypeStruct((B, K), jnp.float32),
        in_specs=specs,
        out_specs=full((B, K)),
    )(*args)


def _tnet(x_bcn, p, k, prefix=()):
    convs = list(prefix) + [p["conv1"], p["conv2"], p["conv3"]]
    relus = (True,) * len(convs)
    pooled = _mlp_maxpool(x_bcn, convs, relus)
    z = _head(pooled, p["fc1"], p["fc2"], p["fc3"], logsm=False)
    B = z.shape[0]
    return z.reshape(B, k, k) + jnp.eye(k, dtype=jnp.float32)[None]


def kernel(x, transform_tnet3_conv1_w, transform_tnet3_conv1_scale, transform_tnet3_conv1_shift, transform_tnet3_conv2_w, transform_tnet3_conv2_scale, transform_tnet3_conv2_shift, transform_tnet3_conv3_w, transform_tnet3_conv3_scale, transform_tnet3_conv3_shift, transform_tnet3_fc1_w, transform_tnet3_fc1_scale, transform_tnet3_fc1_shift, transform_tnet3_fc2_w, transform_tnet3_fc2_scale, transform_tnet3_fc2_shift, transform_tnet3_fc3_w, transform_tnet3_fc3_scale, transform_tnet3_fc3_shift, transform_tnet64_conv1_w, transform_tnet64_conv1_scale, transform_tnet64_conv1_shift, transform_tnet64_conv2_w, transform_tnet64_conv2_scale, transform_tnet64_conv2_shift, transform_tnet64_conv3_w, transform_tnet64_conv3_scale, transform_tnet64_conv3_shift, transform_tnet64_fc1_w, transform_tnet64_fc1_scale, transform_tnet64_fc1_shift, transform_tnet64_fc2_w, transform_tnet64_fc2_scale, transform_tnet64_fc2_shift, transform_tnet64_fc3_w, transform_tnet64_fc3_scale, transform_tnet64_fc3_shift, transform_conv1_w, transform_conv1_scale, transform_conv1_shift, transform_conv2_w, transform_conv2_scale, transform_conv2_shift, transform_conv3_w, transform_conv3_scale, transform_conv3_shift, fc1_w, fc1_scale, fc1_shift, fc2_w, fc2_scale, fc2_shift, fc3_w, fc3_scale, fc3_shift):
    t3 = {
        "conv1": (transform_tnet3_conv1_w, transform_tnet3_conv1_scale, transform_tnet3_conv1_shift),
        "conv2": (transform_tnet3_conv2_w, transform_tnet3_conv2_scale, transform_tnet3_conv2_shift),
        "conv3": (transform_tnet3_conv3_w, transform_tnet3_conv3_scale, transform_tnet3_conv3_shift),
        "fc1": (transform_tnet3_fc1_w, transform_tnet3_fc1_scale, transform_tnet3_fc1_shift),
        "fc2": (transform_tnet3_fc2_w, transform_tnet3_fc2_scale, transform_tnet3_fc2_shift),
        "fc3": (transform_tnet3_fc3_w, transform_tnet3_fc3_scale, transform_tnet3_fc3_shift),
    }
    t64 = {
        "conv1": (transform_tnet64_conv1_w, transform_tnet64_conv1_scale, transform_tnet64_conv1_shift),
        "conv2": (transform_tnet64_conv2_w, transform_tnet64_conv2_scale, transform_tnet64_conv2_shift),
        "conv3": (transform_tnet64_conv3_w, transform_tnet64_conv3_scale, transform_tnet64_conv3_shift),
        "fc1": (transform_tnet64_fc1_w, transform_tnet64_fc1_scale, transform_tnet64_fc1_shift),
        "fc2": (transform_tnet64_fc2_w, transform_tnet64_fc2_scale, transform_tnet64_fc2_shift),
        "fc3": (transform_tnet64_fc3_w, transform_tnet64_fc3_scale, transform_tnet64_fc3_shift),
    }

    m3 = _tnet(x, t3, 3)
    w1_fold = jnp.matmul(m3, transform_conv1_w)
    conv1f = (w1_fold, transform_conv1_scale, transform_conv1_shift)

    m64 = _tnet(x, t64, 64, prefix=(conv1f,))
    w2_fold = jnp.matmul(m64, transform_conv2_w)
    conv2f = (w2_fold, transform_conv2_scale, transform_conv2_shift)

    feat = _mlp_maxpool(
        x, [conv1f, conv2f,
            (transform_conv3_w, transform_conv3_scale, transform_conv3_shift)],
        relus=(True, True, False))
    out = _head(feat, (fc1_w, fc1_scale, fc1_shift),
                (fc2_w, fc2_scale, fc2_shift),
                (fc3_w, fc3_scale, fc3_shift), logsm=True)
    return out, m3, m64


# speedup vs baseline: 2.5731x; 1.1477x over previous
import functools

import jax
import jax.numpy as jnp
from jax import lax
from jax.experimental import pallas as pl
from jax.experimental.pallas import tpu as pltpu

_TN = 4096


def _bf16(w):
    return w.astype(jnp.bfloat16)


def _mlp_max_kernel(x_ref, *refs, per_batch, last_relu, n_layers):
    *prefs, o_ref, acc_ref = refs
    t = pl.program_id(1)

    @pl.when(t == 0)
    def _():
        acc_ref[...] = jnp.full(acc_ref.shape, -jnp.inf, jnp.float32)

    xc = x_ref[0].astype(jnp.bfloat16)
    h = None
    for i in range(n_layers):
        w_ref, b_ref = prefs[2 * i], prefs[2 * i + 1]
        w = w_ref[0] if per_batch[i] else w_ref[...]
        if i == 0:
            h = lax.dot_general(xc, w, (((0,), (0,)), ((), ())),
                                preferred_element_type=jnp.float32)
        else:
            h = jnp.dot(h.astype(jnp.bfloat16), w,
                        preferred_element_type=jnp.float32)
        if i < n_layers - 1:
            h = jnp.maximum(h + b_ref[...], 0.0)

    part = jnp.max(h.reshape(-1, 8, h.shape[-1]), axis=0)
    acc_ref[...] = jnp.maximum(acc_ref[...], part)

    @pl.when(t == pl.num_programs(1) - 1)
    def _():
        r = jnp.max(acc_ref[...], axis=0, keepdims=True) + prefs[-1][...]
        if last_relu:
            r = jnp.maximum(r, 0.0)
        o_ref[0] = r


def _mlp_maxpool(x_bcn, layers, relus):
    B, _, N = x_bcn.shape
    tn = min(N, _TN)
    c_last = layers[-1][0].shape[-1]
    per_batch = tuple(w.ndim == 3 for (w, _, _) in layers)

    in_specs = [pl.BlockSpec((1, 3, tn), lambda b, t: (b, 0, t))]
    args = [x_bcn]
    for w, s, sh in layers:
        wb = _bf16(w * s)
        if wb.ndim == 3:
            in_specs.append(pl.BlockSpec((1,) + wb.shape[1:],
                                         lambda b, t: (b, 0, 0)))
        else:
            in_specs.append(pl.BlockSpec(wb.shape, lambda b, t: (0, 0)))
        in_specs.append(pl.BlockSpec(sh.shape, lambda b, t: (0, 0)))
        args += [wb, sh]

    out = pl.pallas_call(
        functools.partial(_mlp_max_kernel, per_batch=per_batch,
                          last_relu=bool(relus[-1]), n_layers=len(layers)),
        out_shape=jax.ShapeDtypeStruct((B, 1, c_last), jnp.float32),
        grid=(B, N // tn),
        in_specs=in_specs,
        out_specs=pl.BlockSpec((1, 1, c_last), lambda b, t: (b, 0, 0)),
        scratch_shapes=[pltpu.VMEM((8, c_last), jnp.float32)],
        compiler_params=pltpu.CompilerParams(
            dimension_semantics=("parallel", "arbitrary")),
    )(*args)
    return out.reshape(B, c_last)


def _head_kernel(x_ref, w1, s1, b1, w2, s2, b2, w3, s3, b3, o_ref, *, logsm):
    h = jnp.dot(x_ref[...].astype(jnp.bfloat16), w1[...],
                preferred_element_type=jnp.float32)
    h = jnp.maximum(h * s1[...] + b1[...], 0.0)
    h = jnp.dot(h.astype(jnp.bfloat16), w2[...],
                preferred_element_type=jnp.float32)
    h = jnp.maximum(h * s2[...] + b2[...], 0.0)
    z = jnp.dot(h.astype(jnp.bfloat16), w3[...],
                preferred_element_type=jnp.float32)
    z = z * s3[...] + b3[...]
    if logsm:
        z = z - jnp.max(z, axis=-1, keepdims=True)
        z = z - jnp.log(jnp.sum(jnp.exp(z), axis=-1, keepdims=True))
    o_ref[...] = z


def _head(pooled, l1, l2, l3, *, logsm):
    B = pooled.shape[0]
    K = l3[0].shape[-1]
    full = lambda shape: pl.BlockSpec(shape, lambda: (0,) * len(shape))
    args, specs = [pooled], [full(pooled.shape)]
    for w, s, sh in (l1, l2, l3):
        wb = _bf16(w)
        args += [wb, s, sh]
        specs += [full(wb.shape), full(s.shape), full(sh.shape)]
    return pl.pallas_call(
        functools.partial(_head_kernel, logsm=logsm),
        out_shape=jax.ShapeDtypeStruct((B, K), jnp.float32),
        in_specs=specs,
        out_specs=full((B, K)),
    )(*args)


def _tnet(x_bcn, p, k, prefix=()):
    convs = list(prefix) + [p["conv1"], p["conv2"], p["conv3"]]
    relus = (True,) * len(convs)
    pooled = _mlp_maxpool(x_bcn, convs, relus)
    z = _head(pooled, p["fc1"], p["fc2"], p["fc3"], logsm=False)
    B = z.shape[0]
    return z.reshape(B, k, k) + jnp.eye(k, dtype=jnp.float32)[None]


def kernel(x, transform_tnet3_conv1_w, transform_tnet3_conv1_scale, transform_tnet3_conv1_shift, transform_tnet3_conv2_w, transform_tnet3_conv2_scale, transform_tnet3_conv2_shift, transform_tnet3_conv3_w, transform_tnet3_conv3_scale, transform_tnet3_conv3_shift, transform_tnet3_fc1_w, transform_tnet3_fc1_scale, transform_tnet3_fc1_shift, transform_tnet3_fc2_w, transform_tnet3_fc2_scale, transform_tnet3_fc2_shift, transform_tnet3_fc3_w, transform_tnet3_fc3_scale, transform_tnet3_fc3_shift, transform_tnet64_conv1_w, transform_tnet64_conv1_scale, transform_tnet64_conv1_shift, transform_tnet64_conv2_w, transform_tnet64_conv2_scale, transform_tnet64_conv2_shift, transform_tnet64_conv3_w, transform_tnet64_conv3_scale, transform_tnet64_conv3_shift, transform_tnet64_fc1_w, transform_tnet64_fc1_scale, transform_tnet64_fc1_shift, transform_tnet64_fc2_w, transform_tnet64_fc2_scale, transform_tnet64_fc2_shift, transform_tnet64_fc3_w, transform_tnet64_fc3_scale, transform_tnet64_fc3_shift, transform_conv1_w, transform_conv1_scale, transform_conv1_shift, transform_conv2_w, transform_conv2_scale, transform_conv2_shift, transform_conv3_w, transform_conv3_scale, transform_conv3_shift, fc1_w, fc1_scale, fc1_shift, fc2_w, fc2_scale, fc2_shift, fc3_w, fc3_scale, fc3_shift):
    t3 = {
        "conv1": (transform_tnet3_conv1_w, transform_tnet3_conv1_scale, transform_tnet3_conv1_shift),
        "conv2": (transform_tnet3_conv2_w, transform_tnet3_conv2_scale, transform_tnet3_conv2_shift),
        "conv3": (transform_tnet3_conv3_w, transform_tnet3_conv3_scale, transform_tnet3_conv3_shift),
        "fc1": (transform_tnet3_fc1_w, transform_tnet3_fc1_scale, transform_tnet3_fc1_shift),
        "fc2": (transform_tnet3_fc2_w, transform_tnet3_fc2_scale, transform_tnet3_fc2_shift),
        "fc3": (transform_tnet3_fc3_w, transform_tnet3_fc3_scale, transform_tnet3_fc3_shift),
    }
    t64 = {
        "conv1": (transform_tnet64_conv1_w, transform_tnet64_conv1_scale, transform_tnet64_conv1_shift),
        "conv2": (transform_tnet64_conv2_w, transform_tnet64_conv2_scale, transform_tnet64_conv2_shift),
        "conv3": (transform_tnet64_conv3_w, transform_tnet64_conv3_scale, transform_tnet64_conv3_shift),
        "fc1": (transform_tnet64_fc1_w, transform_tnet64_fc1_scale, transform_tnet64_fc1_shift),
        "fc2": (transform_tnet64_fc2_w, transform_tnet64_fc2_scale, transform_tnet64_fc2_shift),
        "fc3": (transform_tnet64_fc3_w, transform_tnet64_fc3_scale, transform_tnet64_fc3_shift),
    }

    m3 = _tnet(x, t3, 3)
    w1_fold = jnp.matmul(m3, transform_conv1_w)
    conv1f = (w1_fold, transform_conv1_scale, transform_conv1_shift)

    m64 = _tnet(x, t64, 64, prefix=(conv1f,))
    w2_fold = jnp.matmul(m64, transform_conv2_w)
    conv2f = (w2_fold, transform_conv2_scale, transform_conv2_shift)

    feat = _mlp_maxpool(
        x, [conv1f, conv2f,
            (transform_conv3_w, transform_conv3_scale, transform_conv3_shift)],
        relus=(True, True, False))
    out = _head(feat, (fc1_w, fc1_scale, fc1_shift),
                (fc2_w, fc2_scale, fc2_shift),
                (fc3_w, fc3_scale, fc3_shift), logsm=True)
    return out, m3, m64


# speedup vs baseline: 2.6648x; 1.0356x over previous
import functools

import jax
import jax.numpy as jnp
from jax import lax
from jax.experimental import pallas as pl
from jax.experimental.pallas import tpu as pltpu

_TN = 4096


def _bf16(w):
    return w.astype(jnp.bfloat16)


def _mlp_max_kernel(x_ref, *refs, per_batch, last_relu, n_layers):
    *prefs, o_ref, acc_ref = refs
    t = pl.program_id(1)

    @pl.when(t == 0)
    def _():
        acc_ref[...] = jnp.full(acc_ref.shape, -jnp.inf, jnp.float32)

    h = x_ref[0].astype(jnp.bfloat16)
    for i in range(n_layers - 1):
        w_ref, b_ref = prefs[2 * i], prefs[2 * i + 1]
        w = w_ref[0] if per_batch[i] else w_ref[...]
        h = lax.dot_general(w, h, (((0,), (0,)), ((), ())),
                            preferred_element_type=jnp.float32)
        h = jnp.maximum(h + b_ref[...], 0.0).astype(jnp.bfloat16)

    w_ref = prefs[2 * (n_layers - 1)]
    wl = w_ref[0] if per_batch[n_layers - 1] else w_ref[...]
    part = acc_ref[...]
    for c in range(0, h.shape[-1], 256):
        y = lax.dot_general(wl, h[:, c:c + 256], (((0,), (0,)), ((), ())),
                            preferred_element_type=jnp.float32)
        part = jnp.maximum(part, jnp.maximum(y[:, :128], y[:, 128:]))
    acc_ref[...] = part

    @pl.when(t == pl.num_programs(1) - 1)
    def _():
        a = acc_ref[...].T
        a = jnp.max(a.reshape(16, 8, a.shape[-1]), axis=0)
        r = jnp.max(a, axis=0, keepdims=True) + prefs[-1][...]
        if last_relu:
            r = jnp.maximum(r, 0.0)
        o_ref[0] = r


def _mlp_maxpool(x_bcn, layers, relus):
    B, _, N = x_bcn.shape
    tn = min(N, _TN)
    c_last = layers[-1][0].shape[-1]
    per_batch = tuple(w.ndim == 3 for (w, _, _) in layers)

    in_specs = [pl.BlockSpec((1, 3, tn), lambda b, t: (b, 0, t))]
    args = [x_bcn]
    for li, (w, s, sh) in enumerate(layers):
        wb = _bf16(w * s)
        if wb.ndim == 3:
            in_specs.append(pl.BlockSpec((1,) + wb.shape[1:],
                                         lambda b, t: (b, 0, 0)))
        else:
            in_specs.append(pl.BlockSpec(wb.shape, lambda b, t: (0, 0)))
        if li < len(layers) - 1:
            sh = sh.reshape(-1, 1)
        in_specs.append(pl.BlockSpec(sh.shape, lambda b, t: (0, 0)))
        args += [wb, sh]

    out = pl.pallas_call(
        functools.partial(_mlp_max_kernel, per_batch=per_batch,
                          last_relu=bool(relus[-1]), n_layers=len(layers)),
        out_shape=jax.ShapeDtypeStruct((B, 1, c_last), jnp.float32),
        grid=(B, N // tn),
        in_specs=in_specs,
        out_specs=pl.BlockSpec((1, 1, c_last), lambda b, t: (b, 0, 0)),
        scratch_shapes=[pltpu.VMEM((c_last, 128), jnp.float32)],
        compiler_params=pltpu.CompilerParams(
            dimension_semantics=("parallel", "arbitrary")),
    )(*args)
    return out.reshape(B, c_last)


def _head_kernel(x_ref, w1, s1, b1, w2, s2, b2, w3, s3, b3, o_ref, *, logsm):
    h = jnp.dot(x_ref[...].astype(jnp.bfloat16), w1[...],
                preferred_element_type=jnp.float32)
    h = jnp.maximum(h * s1[...] + b1[...], 0.0)
    h = jnp.dot(h.astype(jnp.bfloat16), w2[...],
                preferred_element_type=jnp.float32)
    h = jnp.maximum(h * s2[...] + b2[...], 0.0)
    z = jnp.dot(h.astype(jnp.bfloat16), w3[...],
                preferred_element_type=jnp.float32)
    z = z * s3[...] + b3[...]
    if logsm:
        z = z - jnp.max(z, axis=-1, keepdims=True)
        z = z - jnp.log(jnp.sum(jnp.exp(z), axis=-1, keepdims=True))
    o_ref[...] = z


def _head(pooled, l1, l2, l3, *, logsm):
    B = pooled.shape[0]
    K = l3[0].shape[-1]
    full = lambda shape: pl.BlockSpec(shape, lambda: (0,) * len(shape))
    args, specs = [pooled], [full(pooled.shape)]
    for w, s, sh in (l1, l2, l3):
        wb = _bf16(w)
        args += [wb, s, sh]
        specs += [full(wb.shape), full(s.shape), full(sh.shape)]
    return pl.pallas_call(
        functools.partial(_head_kernel, logsm=logsm),
        out_shape=jax.ShapeDtypeStruct((B, K), jnp.float32),
        in_specs=specs,
        out_specs=full((B, K)),
    )(*args)


def _tnet(x_bcn, p, k, prefix=()):
    convs = list(prefix) + [p["conv1"], p["conv2"], p["conv3"]]
    relus = (True,) * len(convs)
    pooled = _mlp_maxpool(x_bcn, convs, relus)
    z = _head(pooled, p["fc1"], p["fc2"], p["fc3"], logsm=False)
    B = z.shape[0]
    return z.reshape(B, k, k) + jnp.eye(k, dtype=jnp.float32)[None]


def kernel(x, transform_tnet3_conv1_w, transform_tnet3_conv1_scale, transform_tnet3_conv1_shift, transform_tnet3_conv2_w, transform_tnet3_conv2_scale, transform_tnet3_conv2_shift, transform_tnet3_conv3_w, transform_tnet3_conv3_scale, transform_tnet3_conv3_shift, transform_tnet3_fc1_w, transform_tnet3_fc1_scale, transform_tnet3_fc1_shift, transform_tnet3_fc2_w, transform_tnet3_fc2_scale, transform_tnet3_fc2_shift, transform_tnet3_fc3_w, transform_tnet3_fc3_scale, transform_tnet3_fc3_shift, transform_tnet64_conv1_w, transform_tnet64_conv1_scale, transform_tnet64_conv1_shift, transform_tnet64_conv2_w, transform_tnet64_conv2_scale, transform_tnet64_conv2_shift, transform_tnet64_conv3_w, transform_tnet64_conv3_scale, transform_tnet64_conv3_shift, transform_tnet64_fc1_w, transform_tnet64_fc1_scale, transform_tnet64_fc1_shift, transform_tnet64_fc2_w, transform_tnet64_fc2_scale, transform_tnet64_fc2_shift, transform_tnet64_fc3_w, transform_tnet64_fc3_scale, transform_tnet64_fc3_shift, transform_conv1_w, transform_conv1_scale, transform_conv1_shift, transform_conv2_w, transform_conv2_scale, transform_conv2_shift, transform_conv3_w, transform_conv3_scale, transform_conv3_shift, fc1_w, fc1_scale, fc1_shift, fc2_w, fc2_scale, fc2_shift, fc3_w, fc3_scale, fc3_shift):
    t3 = {
        "conv1": (transform_tnet3_conv1_w, transform_tnet3_conv1_scale, transform_tnet3_conv1_shift),
        "conv2": (transform_tnet3_conv2_w, transform_tnet3_conv2_scale, transform_tnet3_conv2_shift),
        "conv3": (transform_tnet3_conv3_w, transform_tnet3_conv3_scale, transform_tnet3_conv3_shift),
        "fc1": (transform_tnet3_fc1_w, transform_tnet3_fc1_scale, transform_tnet3_fc1_shift),
        "fc2": (transform_tnet3_fc2_w, transform_tnet3_fc2_scale, transform_tnet3_fc2_shift),
        "fc3": (transform_tnet3_fc3_w, transform_tnet3_fc3_scale, transform_tnet3_fc3_shift),
    }
    t64 = {
        "conv1": (transform_tnet64_conv1_w, transform_tnet64_conv1_scale, transform_tnet64_conv1_shift),
        "conv2": (transform_tnet64_conv2_w, transform_tnet64_conv2_scale, transform_tnet64_conv2_shift),
        "conv3": (transform_tnet64_conv3_w, transform_tnet64_conv3_scale, transform_tnet64_conv3_shift),
        "fc1": (transform_tnet64_fc1_w, transform_tnet64_fc1_scale, transform_tnet64_fc1_shift),
        "fc2": (transform_tnet64_fc2_w, transform_tnet64_fc2_scale, transform_tnet64_fc2_shift),
        "fc3": (transform_tnet64_fc3_w, transform_tnet64_fc3_scale, transform_tnet64_fc3_shift),
    }

    m3 = _tnet(x, t3, 3)
    w1_fold = jnp.matmul(m3, transform_conv1_w)
    conv1f = (w1_fold, transform_conv1_scale, transform_conv1_shift)

    m64 = _tnet(x, t64, 64, prefix=(conv1f,))
    w2_fold = jnp.matmul(m64, transform_conv2_w)
    conv2f = (w2_fold, transform_conv2_scale, transform_conv2_shift)

    feat = _mlp_maxpool(
        x, [conv1f, conv2f,
            (transform_conv3_w, transform_conv3_scale, transform_conv3_shift)],
        relus=(True, True, False))
    out = _head(feat, (fc1_w, fc1_scale, fc1_shift),
                (fc2_w, fc2_scale, fc2_shift),
                (fc3_w, fc3_scale, fc3_shift), logsm=True)
    return out, m3, m64


# speedup vs baseline: 2.8573x; 1.0722x over previous
import functools

import jax
import jax.numpy as jnp
from jax import lax
from jax.experimental import pallas as pl
from jax.experimental.pallas import tpu as pltpu

_TN = 8192


def _bf16(w):
    return w.astype(jnp.bfloat16)


def _mlp_max_kernel(x_ref, *refs, per_batch, last_relu, n_layers):
    *prefs, o_ref, acc_ref = refs
    t = pl.program_id(1)

    @pl.when(t == 0)
    def _():
        acc_ref[...] = jnp.full(acc_ref.shape, -jnp.inf, jnp.float32)

    h = x_ref[0].astype(jnp.bfloat16)
    for i in range(n_layers - 1):
        w_ref, b_ref = prefs[2 * i], prefs[2 * i + 1]
        w = w_ref[0] if per_batch[i] else w_ref[...]
        h = lax.dot_general(w, h, (((0,), (0,)), ((), ())),
                            preferred_element_type=jnp.float32)
        h = jnp.maximum(h + b_ref[...], 0.0).astype(jnp.bfloat16)

    w_ref = prefs[2 * (n_layers - 1)]
    wl = w_ref[0] if per_batch[n_layers - 1] else w_ref[...]
    part = acc_ref[...]
    for c in range(0, h.shape[-1], 256):
        y = lax.dot_general(wl, h[:, c:c + 256], (((0,), (0,)), ((), ())),
                            preferred_element_type=jnp.float32)
        part = jnp.maximum(part, jnp.maximum(y[:, :128], y[:, 128:]))
    acc_ref[...] = part

    @pl.when(t == pl.num_programs(1) - 1)
    def _():
        a = acc_ref[...].T
        a = jnp.max(a.reshape(16, 8, a.shape[-1]), axis=0)
        r = jnp.max(a, axis=0, keepdims=True) + prefs[-1][...]
        if last_relu:
            r = jnp.maximum(r, 0.0)
        o_ref[0] = r


def _mlp_maxpool(x_bcn, layers, relus):
    B, _, N = x_bcn.shape
    tn = min(N, _TN)
    c_last = layers[-1][0].shape[-1]
    per_batch = tuple(w.ndim == 3 for (w, _, _) in layers)

    in_specs = [pl.BlockSpec((1, 3, tn), lambda b, t: (b, 0, t))]
    args = [x_bcn]
    for li, (w, s, sh) in enumerate(layers):
        wb = _bf16(w * s)
        if wb.ndim == 3:
            in_specs.append(pl.BlockSpec((1,) + wb.shape[1:],
                                         lambda b, t: (b, 0, 0)))
        else:
            in_specs.append(pl.BlockSpec(wb.shape, lambda b, t: (0, 0)))
        if li < len(layers) - 1:
            sh = sh.reshape(-1, 1)
        in_specs.append(pl.BlockSpec(sh.shape, lambda b, t: (0, 0)))
        args += [wb, sh]

    out = pl.pallas_call(
        functools.partial(_mlp_max_kernel, per_batch=per_batch,
                          last_relu=bool(relus[-1]), n_layers=len(layers)),
        out_shape=jax.ShapeDtypeStruct((B, 1, c_last), jnp.float32),
        grid=(B, N // tn),
        in_specs=in_specs,
        out_specs=pl.BlockSpec((1, 1, c_last), lambda b, t: (b, 0, 0)),
        scratch_shapes=[pltpu.VMEM((c_last, 128), jnp.float32)],
        compiler_params=pltpu.CompilerParams(
            dimension_semantics=("parallel", "arbitrary")),
    )(*args)
    return out.reshape(B, c_last)


def _head_kernel(x_ref, w1, s1, b1, w2, s2, b2, w3, s3, b3, o_ref, *, logsm):
    h = jnp.dot(x_ref[...].astype(jnp.bfloat16), w1[...],
                preferred_element_type=jnp.float32)
    h = jnp.maximum(h * s1[...] + b1[...], 0.0)
    h = jnp.dot(h.astype(jnp.bfloat16), w2[...],
                preferred_element_type=jnp.float32)
    h = jnp.maximum(h * s2[...] + b2[...], 0.0)
    z = jnp.dot(h.astype(jnp.bfloat16), w3[...],
                preferred_element_type=jnp.float32)
    z = z * s3[...] + b3[...]
    if logsm:
        z = z - jnp.max(z, axis=-1, keepdims=True)
        z = z - jnp.log(jnp.sum(jnp.exp(z), axis=-1, keepdims=True))
    o_ref[...] = z


def _head(pooled, l1, l2, l3, *, logsm):
    B = pooled.shape[0]
    K = l3[0].shape[-1]
    full = lambda shape: pl.BlockSpec(shape, lambda: (0,) * len(shape))
    args, specs = [pooled], [full(pooled.shape)]
    for w, s, sh in (l1, l2, l3):
        wb = _bf16(w)
        args += [wb, s, sh]
        specs += [full(wb.shape), full(s.shape), full(sh.shape)]
    return pl.pallas_call(
        functools.partial(_head_kernel, logsm=logsm),
        out_shape=jax.ShapeDtypeStruct((B, K), jnp.float32),
        in_specs=specs,
        out_specs=full((B, K)),
    )(*args)


def _tnet(x_bcn, p, k, prefix=()):
    convs = list(prefix) + [p["conv1"], p["conv2"], p["conv3"]]
    relus = (True,) * len(convs)
    pooled = _mlp_maxpool(x_bcn, convs, relus)
    z = _head(pooled, p["fc1"], p["fc2"], p["fc3"], logsm=False)
    B = z.shape[0]
    return z.reshape(B, k, k) + jnp.eye(k, dtype=jnp.float32)[None]


def kernel(x, transform_tnet3_conv1_w, transform_tnet3_conv1_scale, transform_tnet3_conv1_shift, transform_tnet3_conv2_w, transform_tnet3_conv2_scale, transform_tnet3_conv2_shift, transform_tnet3_conv3_w, transform_tnet3_conv3_scale, transform_tnet3_conv3_shift, transform_tnet3_fc1_w, transform_tnet3_fc1_scale, transform_tnet3_fc1_shift, transform_tnet3_fc2_w, transform_tnet3_fc2_scale, transform_tnet3_fc2_shift, transform_tnet3_fc3_w, transform_tnet3_fc3_scale, transform_tnet3_fc3_shift, transform_tnet64_conv1_w, transform_tnet64_conv1_scale, transform_tnet64_conv1_shift, transform_tnet64_conv2_w, transform_tnet64_conv2_scale, transform_tnet64_conv2_shift, transform_tnet64_conv3_w, transform_tnet64_conv3_scale, transform_tnet64_conv3_shift, transform_tnet64_fc1_w, transform_tnet64_fc1_scale, transform_tnet64_fc1_shift, transform_tnet64_fc2_w, transform_tnet64_fc2_scale, transform_tnet64_fc2_shift, transform_tnet64_fc3_w, transform_tnet64_fc3_scale, transform_tnet64_fc3_shift, transform_conv1_w, transform_conv1_scale, transform_conv1_shift, transform_conv2_w, transform_conv2_scale, transform_conv2_shift, transform_conv3_w, transform_conv3_scale, transform_conv3_shift, fc1_w, fc1_scale, fc1_shift, fc2_w, fc2_scale, fc2_shift, fc3_w, fc3_scale, fc3_shift):
    t3 = {
        "conv1": (transform_tnet3_conv1_w, transform_tnet3_conv1_scale, transform_tnet3_conv1_shift),
        "conv2": (transform_tnet3_conv2_w, transform_tnet3_conv2_scale, transform_tnet3_conv2_shift),
        "conv3": (transform_tnet3_conv3_w, transform_tnet3_conv3_scale, transform_tnet3_conv3_shift),
        "fc1": (transform_tnet3_fc1_w, transform_tnet3_fc1_scale, transform_tnet3_fc1_shift),
        "fc2": (transform_tnet3_fc2_w, transform_tnet3_fc2_scale, transform_tnet3_fc2_shift),
        "fc3": (transform_tnet3_fc3_w, transform_tnet3_fc3_scale, transform_tnet3_fc3_shift),
    }
    t64 = {
        "conv1": (transform_tnet64_conv1_w, transform_tnet64_conv1_scale, transform_tnet64_conv1_shift),
        "conv2": (transform_tnet64_conv2_w, transform_tnet64_conv2_scale, transform_tnet64_conv2_shift),
        "conv3": (transform_tnet64_conv3_w, transform_tnet64_conv3_scale, transform_tnet64_conv3_shift),
        "fc1": (transform_tnet64_fc1_w, transform_tnet64_fc1_scale, transform_tnet64_fc1_shift),
        "fc2": (transform_tnet64_fc2_w, transform_tnet64_fc2_scale, transform_tnet64_fc2_shift),
        "fc3": (transform_tnet64_fc3_w, transform_tnet64_fc3_scale, transform_tnet64_fc3_shift),
    }

    m3 = _tnet(x, t3, 3)
    w1_fold = jnp.matmul(m3, transform_conv1_w)
    conv1f = (w1_fold, transform_conv1_scale, transform_conv1_shift)

    m64 = _tnet(x, t64, 64, prefix=(conv1f,))
    w2_fold = jnp.matmul(m64, transform_conv2_w)
    conv2f = (w2_fold, transform_conv2_scale, transform_conv2_shift)

    feat = _mlp_maxpool(
        x, [conv1f, conv2f,
            (transform_conv3_w, transform_conv3_scale, transform_conv3_shift)],
        relus=(True, True, False))
    out = _head(feat, (fc1_w, fc1_scale, fc1_shift),
                (fc2_w, fc2_scale, fc2_shift),
                (fc3_w, fc3_scale, fc3_shift), logsm=True)
    return out, m3, m64


# speedup vs baseline: 2.8603x; 1.0011x over previous
import functools

import jax
import jax.numpy as jnp
from jax import lax
from jax.experimental import pallas as pl
from jax.experimental.pallas import tpu as pltpu

_TN = 8192


def _bf16(w):
    return w.astype(jnp.bfloat16)


def _mlp_max_kernel(x_ref, *refs, per_batch, last_relu, n_layers):
    *prefs, o_ref = refs
    xin = x_ref[0].astype(jnp.bfloat16)
    wl_ref = prefs[2 * (n_layers - 1)]
    wl = wl_ref[0] if per_batch[n_layers - 1] else wl_ref[...]
    c_last = wl.shape[-1]

    part = jnp.full((c_last, 128), -jnp.inf, jnp.float32)
    h = xin
    for i in range(n_layers - 1):
        w_ref, b_ref = prefs[2 * i], prefs[2 * i + 1]
        w = w_ref[0] if per_batch[i] else w_ref[...]
        h = lax.dot_general(w, h, (((0,), (0,)), ((), ())),
                            preferred_element_type=jnp.float32)
        h = jnp.maximum(h + b_ref[...], 0.0).astype(jnp.bfloat16)
    for c in range(0, h.shape[-1], 256):
        y = lax.dot_general(wl, h[:, c:c + 256], (((0,), (0,)), ((), ())),
                            preferred_element_type=jnp.float32)
        part = jnp.maximum(part, jnp.maximum(y[:, :128], y[:, 128:]))

    a = part.T
    a = jnp.max(a.reshape(16, 8, a.shape[-1]), axis=0)
    r = jnp.max(a, axis=0, keepdims=True) + prefs[-1][...]
    if last_relu:
        r = jnp.maximum(r, 0.0)
    o_ref[0] = r


def _mlp_maxpool(x_bcn, layers, relus):
    B, _, N = x_bcn.shape
    c_last = layers[-1][0].shape[-1]
    per_batch = tuple(w.ndim == 3 for (w, _, _) in layers)

    in_specs = [pl.BlockSpec((1, 3, N), lambda b: (b, 0, 0))]
    args = [x_bcn]
    for li, (w, s, sh) in enumerate(layers):
        wb = _bf16(w * s)
        if wb.ndim == 3:
            in_specs.append(pl.BlockSpec((1,) + wb.shape[1:],
                                         lambda b: (b, 0, 0)))
        else:
            in_specs.append(pl.BlockSpec(wb.shape, lambda b: (0, 0)))
        if li < len(layers) - 1:
            sh = sh.reshape(-1, 1)
        in_specs.append(pl.BlockSpec(sh.shape, lambda b: (0, 0)))
        args += [wb, sh]

    out = pl.pallas_call(
        functools.partial(_mlp_max_kernel, per_batch=per_batch,
                          last_relu=bool(relus[-1]), n_layers=len(layers)),
        out_shape=jax.ShapeDtypeStruct((B, 1, c_last), jnp.float32),
        grid=(B,),
        in_specs=in_specs,
        out_specs=pl.BlockSpec((1, 1, c_last), lambda b: (b, 0, 0)),
        compiler_params=pltpu.CompilerParams(
            dimension_semantics=("parallel",)),
    )(*args)
    return out.reshape(B, c_last)


def _head_kernel(x_ref, w1, s1, b1, w2, s2, b2, w3, s3, b3, o_ref, *, logsm):
    h = jnp.dot(x_ref[...].astype(jnp.bfloat16), w1[...],
                preferred_element_type=jnp.float32)
    h = jnp.maximum(h * s1[...] + b1[...], 0.0)
    h = jnp.dot(h.astype(jnp.bfloat16), w2[...],
                preferred_element_type=jnp.float32)
    h = jnp.maximum(h * s2[...] + b2[...], 0.0)
    z = jnp.dot(h.astype(jnp.bfloat16), w3[...],
                preferred_element_type=jnp.float32)
    z = z * s3[...] + b3[...]
    if logsm:
        z = z - jnp.max(z, axis=-1, keepdims=True)
        z = z - jnp.log(jnp.sum(jnp.exp(z), axis=-1, keepdims=True))
    o_ref[...] = z


def _head(pooled, l1, l2, l3, *, logsm):
    B = pooled.shape[0]
    K = l3[0].shape[-1]
    full = lambda shape: pl.BlockSpec(shape, lambda: (0,) * len(shape))
    args, specs = [pooled], [full(pooled.shape)]
    for w, s, sh in (l1, l2, l3):
        wb = _bf16(w)
        args += [wb, s, sh]
        specs += [full(wb.shape), full(s.shape), full(sh.shape)]
    return pl.pallas_call(
        functools.partial(_head_kernel, logsm=logsm),
        out_shape=jax.ShapeDtypeStruct((B, K), jnp.float32),
        in_specs=specs,
        out_specs=full((B, K)),
    )(*args)


def _tnet(x_bcn, p, k, prefix=()):
    convs = list(prefix) + [p["conv1"], p["conv2"], p["conv3"]]
    relus = (True,) * len(convs)
    pooled = _mlp_maxpool(x_bcn, convs, relus)
    z = _head(pooled, p["fc1"], p["fc2"], p["fc3"], logsm=False)
    B = z.shape[0]
    return z.reshape(B, k, k) + jnp.eye(k, dtype=jnp.float32)[None]


def kernel(x, transform_tnet3_conv1_w, transform_tnet3_conv1_scale, transform_tnet3_conv1_shift, transform_tnet3_conv2_w, transform_tnet3_conv2_scale, transform_tnet3_conv2_shift, transform_tnet3_conv3_w, transform_tnet3_conv3_scale, transform_tnet3_conv3_shift, transform_tnet3_fc1_w, transform_tnet3_fc1_scale, transform_tnet3_fc1_shift, transform_tnet3_fc2_w, transform_tnet3_fc2_scale, transform_tnet3_fc2_shift, transform_tnet3_fc3_w, transform_tnet3_fc3_scale, transform_tnet3_fc3_shift, transform_tnet64_conv1_w, transform_tnet64_conv1_scale, transform_tnet64_conv1_shift, transform_tnet64_conv2_w, transform_tnet64_conv2_scale, transform_tnet64_conv2_shift, transform_tnet64_conv3_w, transform_tnet64_conv3_scale, transform_tnet64_conv3_shift, transform_tnet64_fc1_w, transform_tnet64_fc1_scale, transform_tnet64_fc1_shift, transform_tnet64_fc2_w, transform_tnet64_fc2_scale, transform_tnet64_fc2_shift, transform_tnet64_fc3_w, transform_tnet64_fc3_scale, transform_tnet64_fc3_shift, transform_conv1_w, transform_conv1_scale, transform_conv1_shift, transform_conv2_w, transform_conv2_scale, transform_conv2_shift, transform_conv3_w, transform_conv3_scale, transform_conv3_shift, fc1_w, fc1_scale, fc1_shift, fc2_w, fc2_scale, fc2_shift, fc3_w, fc3_scale, fc3_shift):
    t3 = {
        "conv1": (transform_tnet3_conv1_w, transform_tnet3_conv1_scale, transform_tnet3_conv1_shift),
        "conv2": (transform_tnet3_conv2_w, transform_tnet3_conv2_scale, transform_tnet3_conv2_shift),
        "conv3": (transform_tnet3_conv3_w, transform_tnet3_conv3_scale, transform_tnet3_conv3_shift),
        "fc1": (transform_tnet3_fc1_w, transform_tnet3_fc1_scale, transform_tnet3_fc1_shift),
        "fc2": (transform_tnet3_fc2_w, transform_tnet3_fc2_scale, transform_tnet3_fc2_shift),
        "fc3": (transform_tnet3_fc3_w, transform_tnet3_fc3_scale, transform_tnet3_fc3_shift),
    }
    t64 = {
        "conv1": (transform_tnet64_conv1_w, transform_tnet64_conv1_scale, transform_tnet64_conv1_shift),
        "conv2": (transform_tnet64_conv2_w, transform_tnet64_conv2_scale, transform_tnet64_conv2_shift),
        "conv3": (transform_tnet64_conv3_w, transform_tnet64_conv3_scale, transform_tnet64_conv3_shift),
        "fc1": (transform_tnet64_fc1_w, transform_tnet64_fc1_scale, transform_tnet64_fc1_shift),
        "fc2": (transform_tnet64_fc2_w, transform_tnet64_fc2_scale, transform_tnet64_fc2_shift),
        "fc3": (transform_tnet64_fc3_w, transform_tnet64_fc3_scale, transform_tnet64_fc3_shift),
    }

    m3 = _tnet(x, t3, 3)
    w1_fold = jnp.matmul(m3, transform_conv1_w)
    conv1f = (w1_fold, transform_conv1_scale, transform_conv1_shift)

    m64 = _tnet(x, t64, 64, prefix=(conv1f,))
    w2_fold = jnp.matmul(m64, transform_conv2_w)
    conv2f = (w2_fold, transform_conv2_scale, transform_conv2_shift)

    feat = _mlp_maxpool(
        x, [conv1f, conv2f,
            (transform_conv3_w, transform_conv3_scale, transform_conv3_shift)],
        relus=(True, True, False))
    out = _head(feat, (fc1_w, fc1_scale, fc1_shift),
                (fc2_w, fc2_scale, fc2_shift),
                (fc3_w, fc3_scale, fc3_shift), logsm=True)
    return out, m3, m64


# speedup vs baseline: 3.8791x; 1.3562x over previous
import functools

import jax
import jax.numpy as jnp
from jax import lax
from jax.experimental import pallas as pl
from jax.experimental.pallas import tpu as pltpu

_TN = 8192


def _bf16(w):
    return w.astype(jnp.bfloat16)


def _mlp_max_kernel(x_ref, *refs, per_batch, last_relu, n_layers):
    *prefs, o_ref = refs
    xin = x_ref[0].astype(jnp.bfloat16)
    wl_ref = prefs[2 * (n_layers - 1)]
    wl = wl_ref[0] if per_batch[n_layers - 1] else wl_ref[...]
    c_last = wl.shape[-1]

    part = jnp.full((c_last, 128), -jnp.inf, jnp.float32)
    h = xin
    for i in range(n_layers - 1):
        w_ref, b_ref = prefs[2 * i], prefs[2 * i + 1]
        w = w_ref[0] if per_batch[i] else w_ref[...]
        h = lax.dot_general(w, h, (((0,), (0,)), ((), ())),
                            preferred_element_type=jnp.float32)
        h = jnp.maximum(h + b_ref[...], 0.0).astype(jnp.bfloat16)
    for c in range(0, h.shape[-1], 256):
        y = lax.dot_general(wl, h[:, c:c + 256], (((0,), (0,)), ((), ())),
                            preferred_element_type=jnp.float32)
        part = jnp.maximum(part, jnp.maximum(y[:, :128], y[:, 128:]))

    a = part.T
    a = jnp.max(a.reshape(16, 8, a.shape[-1]), axis=0)
    r = jnp.max(a, axis=0, keepdims=True) + prefs[-1][...]
    if last_relu:
        r = jnp.maximum(r, 0.0)
    o_ref[0] = r


def _mlp_maxpool(x_bcn, layers, relus):
    B, _, N = x_bcn.shape
    c_last = layers[-1][0].shape[-1]
    per_batch = tuple(w.ndim == 3 for (w, _, _) in layers)

    in_specs = [pl.BlockSpec((1, 3, N), lambda b: (b, 0, 0))]
    args = [x_bcn]
    for li, (w, s, sh) in enumerate(layers):
        wb = _bf16(w * s)
        if wb.ndim == 3:
            in_specs.append(pl.BlockSpec((1,) + wb.shape[1:],
                                         lambda b: (b, 0, 0)))
        else:
            in_specs.append(pl.BlockSpec(wb.shape, lambda b: (0, 0)))
        if li < len(layers) - 1:
            sh = sh.reshape(-1, 1)
        in_specs.append(pl.BlockSpec(sh.shape, lambda b: (0, 0)))
        args += [wb, sh]

    out = pl.pallas_call(
        functools.partial(_mlp_max_kernel, per_batch=per_batch,
                          last_relu=bool(relus[-1]), n_layers=len(layers)),
        out_shape=jax.ShapeDtypeStruct((B, 1, c_last), jnp.float32),
        grid=(B,),
        in_specs=in_specs,
        out_specs=pl.BlockSpec((1, 1, c_last), lambda b: (b, 0, 0)),
        compiler_params=pltpu.CompilerParams(
            dimension_semantics=("parallel",)),
    )(*args)
    return out.reshape(B, c_last)


def _head_kernel(x_ref, w1, s1, b1, w2, s2, b2, w3, s3, b3, o_ref, *, logsm):
    h = jnp.dot(x_ref[...].astype(jnp.bfloat16), w1[...],
                preferred_element_type=jnp.float32)
    h = jnp.maximum(h * s1[...] + b1[...], 0.0)
    h = jnp.dot(h.astype(jnp.bfloat16), w2[...],
                preferred_element_type=jnp.float32)
    h = jnp.maximum(h * s2[...] + b2[...], 0.0)
    z = jnp.dot(h.astype(jnp.bfloat16), w3[...],
                preferred_element_type=jnp.float32)
    z = z * s3[...] + b3[...]
    if logsm:
        z = z - jnp.max(z, axis=-1, keepdims=True)
        z = z - jnp.log(jnp.sum(jnp.exp(z), axis=-1, keepdims=True))
    o_ref[...] = z


def _head(pooled, l1, l2, l3, *, logsm):
    B = pooled.shape[0]
    K = l3[0].shape[-1]
    full = lambda shape: pl.BlockSpec(shape, lambda: (0,) * len(shape))
    args, specs = [pooled], [full(pooled.shape)]
    for w, s, sh in (l1, l2, l3):
        wb = _bf16(w)
        args += [wb, s, sh]
        specs += [full(wb.shape), full(s.shape), full(sh.shape)]
    return pl.pallas_call(
        functools.partial(_head_kernel, logsm=logsm),
        out_shape=jax.ShapeDtypeStruct((B, K), jnp.float32),
        in_specs=specs,
        out_specs=full((B, K)),
    )(*args)


def _tnet(x_bcn, p, k, prefix=()):
    convs = list(prefix) + [p["conv1"], p["conv2"], p["conv3"]]
    relus = (True,) * len(convs)
    pooled = _mlp_maxpool(x_bcn, convs, relus)
    z = _head(pooled, p["fc1"], p["fc2"], p["fc3"], logsm=False)
    B = z.shape[0]
    return z.reshape(B, k, k) + jnp.eye(k, dtype=jnp.float32)[None]


def _forward(x, transform_tnet3_conv1_w, transform_tnet3_conv1_scale, transform_tnet3_conv1_shift, transform_tnet3_conv2_w, transform_tnet3_conv2_scale, transform_tnet3_conv2_shift, transform_tnet3_conv3_w, transform_tnet3_conv3_scale, transform_tnet3_conv3_shift, transform_tnet3_fc1_w, transform_tnet3_fc1_scale, transform_tnet3_fc1_shift, transform_tnet3_fc2_w, transform_tnet3_fc2_scale, transform_tnet3_fc2_shift, transform_tnet3_fc3_w, transform_tnet3_fc3_scale, transform_tnet3_fc3_shift, transform_tnet64_conv1_w, transform_tnet64_conv1_scale, transform_tnet64_conv1_shift, transform_tnet64_conv2_w, transform_tnet64_conv2_scale, transform_tnet64_conv2_shift, transform_tnet64_conv3_w, transform_tnet64_conv3_scale, transform_tnet64_conv3_shift, transform_tnet64_fc1_w, transform_tnet64_fc1_scale, transform_tnet64_fc1_shift, transform_tnet64_fc2_w, transform_tnet64_fc2_scale, transform_tnet64_fc2_shift, transform_tnet64_fc3_w, transform_tnet64_fc3_scale, transform_tnet64_fc3_shift, transform_conv1_w, transform_conv1_scale, transform_conv1_shift, transform_conv2_w, transform_conv2_scale, transform_conv2_shift, transform_conv3_w, transform_conv3_scale, transform_conv3_shift, fc1_w, fc1_scale, fc1_shift, fc2_w, fc2_scale, fc2_shift, fc3_w, fc3_scale, fc3_shift):
    t3 = {
        "conv1": (transform_tnet3_conv1_w, transform_tnet3_conv1_scale, transform_tnet3_conv1_shift),
        "conv2": (transform_tnet3_conv2_w, transform_tnet3_conv2_scale, transform_tnet3_conv2_shift),
        "conv3": (transform_tnet3_conv3_w, transform_tnet3_conv3_scale, transform_tnet3_conv3_shift),
        "fc1": (transform_tnet3_fc1_w, transform_tnet3_fc1_scale, transform_tnet3_fc1_shift),
        "fc2": (transform_tnet3_fc2_w, transform_tnet3_fc2_scale, transform_tnet3_fc2_shift),
        "fc3": (transform_tnet3_fc3_w, transform_tnet3_fc3_scale, transform_tnet3_fc3_shift),
    }
    t64 = {
        "conv1": (transform_tnet64_conv1_w, transform_tnet64_conv1_scale, transform_tnet64_conv1_shift),
        "conv2": (transform_tnet64_conv2_w, transform_tnet64_conv2_scale, transform_tnet64_conv2_shift),
        "conv3": (transform_tnet64_conv3_w, transform_tnet64_conv3_scale, transform_tnet64_conv3_shift),
        "fc1": (transform_tnet64_fc1_w, transform_tnet64_fc1_scale, transform_tnet64_fc1_shift),
        "fc2": (transform_tnet64_fc2_w, transform_tnet64_fc2_scale, transform_tnet64_fc2_shift),
        "fc3": (transform_tnet64_fc3_w, transform_tnet64_fc3_scale, transform_tnet64_fc3_shift),
    }

    m3 = _tnet(x, t3, 3)
    w1_fold = jnp.matmul(m3, transform_conv1_w)
    conv1f = (w1_fold, transform_conv1_scale, transform_conv1_shift)

    m64 = _tnet(x, t64, 64, prefix=(conv1f,))
    w2_fold = jnp.matmul(m64, transform_conv2_w)
    conv2f = (w2_fold, transform_conv2_scale, transform_conv2_shift)

    feat = _mlp_maxpool(
        x, [conv1f, conv2f,
            (transform_conv3_w, transform_conv3_scale, transform_conv3_shift)],
        relus=(True, True, False))
    out = _head(feat, (fc1_w, fc1_scale, fc1_shift),
                (fc2_w, fc2_scale, fc2_shift),
                (fc3_w, fc3_scale, fc3_shift), logsm=True)
    return out, m3, m64


def kernel(*args):
    x = args[0]
    devs = jax.devices()
    nd = 2 if len(devs) >= 2 and x.shape[0] % 2 == 0 else 1
    if nd == 1:
        return _forward(*args)
    mesh = jax.sharding.Mesh(devs[:nd], ("d",))
    P = jax.sharding.PartitionSpec
    in_specs = (P("d"),) + (P(),) * (len(args) - 1)
    out_specs = (P("d"), P("d"), P("d"))
    return jax.shard_map(_forward, mesh=mesh, in_specs=in_specs,
                         out_specs=out_specs, check_vma=False)(*args)
